# baseline (speedup 1.0000x reference)
import functools

import numpy as np
import jax
import jax.numpy as jnp
from jax.experimental import pallas as pl
from jax.experimental.pallas import tpu as pltpu

B = 512
A = 32
DA = 8
DB = 8
H = 4
HD = 32
F = H * HD
LAT = 64

G = 512
GA = G * A


def _leaky(x):
    return jnp.where(x >= 0, x, 0.2 * x)


def _per_head_rows(es):
    es3 = es.reshape(G, A, H)
    est = jnp.transpose(es3, (0, 2, 1))
    return [est[:, h:h + 1, :].reshape(G, A) for h in range(H)]


def _fused_kernel(sig_ref, acts_ref, actsw_ref, w1a_ref, w1b_ref,
                  as1_ref, ad1_ref, as2_ref, ad2_ref,
                  b1_ref, w2_ref, b2_ref,
                  wm_ref, bm_ref, wv_ref, bv_ref, wa_ref, ba_ref,
                  wb_ref, bb_ref, hm_ref, exp_ref, hsel_ref,
                  c1_ref, t1_ref, e0_ref, eyef_ref,
                  act_out, mu_out, lv_out, bel_out):
    f32 = jnp.float32
    dot = functools.partial(jnp.dot, preferred_element_type=f32)

    sig = sig_ref[...].reshape(G, DB)
    acts = acts_ref[...]
    hm = hm_ref[...]
    hsel = hsel_ref[...]
    eyef = eyef_ref[...]

    v1 = dot(w1b_ref[...] * as1_ref[...], hsel)
    w1aas1 = dot(w1a_ref[...] * as1_ref[...], hsel)
    kq = dot(dot(c1_ref[...], v1), exp_ref[...]) * t1_ref[...]
    ad1m = dot(eyef * ad1_ref[...], hm)
    a2m = jnp.concatenate([dot(eyef * as2_ref[...], hsel),
                           dot(eyef * ad2_ref[...], hsel)], axis=1)

    sigw = dot(sig, w1a_ref[...])
    bel_rows = jnp.concatenate(
        [sigw.reshape(G, 1, F), jnp.zeros((G, A - 1, F), f32)],
        axis=1).reshape(GA, F)
    h1 = bel_rows + dot(acts, w1b_ref[...])

    q = dot(actsw_ref[...], kq) \
        + dot(dot(sig, w1aas1), e0_ref[...])

    es_b = jnp.broadcast_to(q.reshape(G, 1, F), (G, A, F)).reshape(GA, F)
    ed_b = dot(h1, ad1m)
    s = _leaky(es_b + ed_b)
    m = jnp.max(s, axis=1, keepdims=True)
    ex = jnp.exp(s - m)
    denom = dot(ex, hm) + 1e-16
    alpha_pack = ex / denom

    outs = []
    for g in range(G):
        hg = h1[g * A:(g + 1) * A, :]
        hp = jnp.concatenate([hg, hg, hg, hg], axis=0) * hm
        outs.append(dot(alpha_pack[g * A:(g + 1) * A, :], hp))
    x1 = jax.nn.relu(jnp.concatenate(outs, axis=0) + b1_ref[...])

    h2 = dot(x1, w2_ref[...])
    e2 = dot(h2, a2m)
    es2 = e2[:, :H]
    q2 = jnp.concatenate(_per_head_rows(es2), axis=1)
    ed2 = e2[:, H:]
    ed2a = ed2.reshape(G, A, H)[:, 0:1, :].reshape(G, H)
    s2 = _leaky(q2 + dot(ed2a, exp_ref[...]))
    m2 = jnp.max(s2, axis=1, keepdims=True)
    ex2 = jnp.exp(s2 - m2)
    den2 = dot(ex2, hm) + 1e-16
    a2 = ex2 / den2
    stack = jnp.concatenate(
        [a2[:, h * A:(h + 1) * A].reshape(G, 1, A) for h in range(H)],
        axis=1)
    node_w = jnp.transpose(stack, (0, 2, 1)).reshape(GA, H)
    wh2 = h2 * dot(node_w, exp_ref[...])
    agent = jax.nn.relu(jnp.sum(wh2.reshape(G, A, F), axis=1)
                        + b2_ref[...])

    mu = dot(agent, wm_ref[...]) + bm_ref[...]
    lv = dot(agent, wv_ref[...]) + bv_ref[...]
    act = dot(mu, wa_ref[...]) + ba_ref[...]
    bel = dot(agent, wb_ref[...]) + bb_ref[...]

    act_out[...] = act[None]
    mu_out[...] = mu[None]
    lv_out[...] = lv[None]
    bel_out[...] = bel[None]


@jax.jit
def kernel(signals, neighbor_actions, W1, att_src1, att_dst1, b1,
           W2, att_src2, att_dst2, b2, Wm, bm, Wv, bv, Wa, ba, Wb, bb):
    acts_r = neighbor_actions.reshape(B * A, DA)
    acts_w = neighbor_actions.reshape(B, A * DA)
    w1a, w1b = W1[:DB], W1[DB:]

    hm = jnp.asarray(np.kron(np.eye(H, dtype=np.float32),
                             np.ones((HD, HD), np.float32)))
    expd = jnp.asarray(np.kron(np.eye(H, dtype=np.float32),
                               np.ones((1, A), np.float32)))
    hsel = jnp.asarray(np.kron(np.eye(H, dtype=np.float32),
                               np.ones((HD, 1), np.float32)))
    c1 = jnp.asarray(np.kron(np.ones((A, 1), np.float32),
                             np.eye(DA, dtype=np.float32)))
    t1 = jnp.asarray(np.tile(np.kron(np.eye(A, dtype=np.float32),
                                     np.ones((DA, 1), np.float32)),
                             (1, H)))
    e0 = jnp.asarray(np.kron(np.eye(H, dtype=np.float32),
                             np.eye(1, A, dtype=np.float32)))
    eyef = jnp.asarray(np.eye(F, dtype=np.float32))

    full = lambda shape: pl.BlockSpec(shape, lambda i: (0, 0))
    tile = lambda shape: pl.BlockSpec(shape, lambda i: (i, 0))
    tile3 = lambda w: pl.BlockSpec((1, G, w), lambda i: (i, 0, 0))

    NB = B // G
    out_shapes = (
        jax.ShapeDtypeStruct((NB, G, DA * A), jnp.float32),
        jax.ShapeDtypeStruct((NB, G, LAT), jnp.float32),
        jax.ShapeDtypeStruct((NB, G, LAT), jnp.float32),
        jax.ShapeDtypeStruct((NB, G, DB), jnp.float32),
    )
    grid = (B // G,)
    res = pl.pallas_call(
        _fused_kernel,
        grid=grid,
        in_specs=[
            tile3(DB),
            tile((GA, DA)),
            tile((G, A * DA)),
            full((DB, F)),
            full((DA, F)),
            full((1, F)),
            full((1, F)),
            full((1, F)),
            full((1, F)),
            full((1, F)),
            full((F, F)),
            full((1, F)),
            full((F, LAT)),
            full((1, LAT)),
            full((F, LAT)),
            full((1, LAT)),
            full((LAT, DA * A)),
            full((1, DA * A)),
            full((F, DB)),
            full((1, DB)),
            full((F, F)),
            full((H, H * A)),
            full((F, H)),
            full((A * DA, DA)),
            full((A * DA, H * A)),
            full((H, H * A)),
            full((F, F)),
        ],
        out_specs=(
            tile3(DA * A),
            tile3(LAT),
            tile3(LAT),
            tile3(DB),
        ),
        out_shape=out_shapes,
        compiler_params=pltpu.CompilerParams(
            dimension_semantics=("parallel",),
        ),
    )(signals.reshape(NB, G, DB), acts_r, acts_w, w1a, w1b,
      att_src1.reshape(1, F), att_dst1.reshape(1, F),
      att_src2.reshape(1, F), att_dst2.reshape(1, F),
      b1.reshape(1, F), W2, b2.reshape(1, F), Wm,
      bm.reshape(1, LAT), Wv, bv.reshape(1, LAT), Wa,
      ba.reshape(1, DA * A), Wb, bb.reshape(1, DB), hm, expd, hsel,
      c1, t1, e0, eyef)
    act, mu, lv, bel = res
    return (act.reshape(B, DA * A), mu.reshape(B, LAT),
            lv.reshape(B, LAT), bel.reshape(B, DB))


# speedup vs baseline: 1.0358x; 1.0358x over previous
import functools

import numpy as np
import jax
import jax.numpy as jnp
from jax.experimental import pallas as pl
from jax.experimental.pallas import tpu as pltpu

B = 512
A = 32
DA = 8
DB = 8
H = 4
HD = 32
F = H * HD
LAT = 64

G = 512
GA = G * A


def _leaky(x):
    return jnp.where(x >= 0, x, 0.2 * x)


def _per_head_rows(es):
    es3 = es.reshape(G, A, H)
    est = jnp.transpose(es3, (0, 2, 1))
    return [est[:, h:h + 1, :].reshape(G, A) for h in range(H)]


def _fused_kernel(sig_ref, acts_ref, actsw_ref, w1a_ref, w1b_ref,
                  as1_ref, ad1_ref, as2_ref, ad2_ref,
                  b1_ref, w2_ref, b2_ref,
                  wm_ref, bm_ref, wv_ref, bv_ref, wa_ref, ba_ref,
                  wb_ref, bb_ref, hm_ref, exp_ref, hsel_ref,
                  c1_ref, t1_ref, e0_ref, eyef_ref,
                  act_out, mu_out, lv_out, bel_out):
    f32 = jnp.float32
    dot = functools.partial(jnp.dot, preferred_element_type=f32)

    sig = sig_ref[...].reshape(G, DB)
    acts = acts_ref[...]
    hm = hm_ref[...]
    hsel = hsel_ref[...]
    eyef = eyef_ref[...]

    v1 = dot(w1b_ref[...] * as1_ref[...], hsel)
    w1aas1 = dot(w1a_ref[...] * as1_ref[...], hsel)
    kq = dot(dot(c1_ref[...], v1), exp_ref[...]) * t1_ref[...]
    ad1m = dot(eyef * ad1_ref[...], hm)
    as2m = dot(eyef * as2_ref[...], hsel)
    ad2m = dot(eyef * ad2_ref[...], hsel)

    sigw = dot(sig, w1a_ref[...])
    bel_rows = jnp.concatenate(
        [sigw.reshape(G, 1, F), jnp.zeros((G, A - 1, F), f32)],
        axis=1).reshape(GA, F)
    h1 = bel_rows + dot(acts, w1b_ref[...])

    q = dot(actsw_ref[...], kq) \
        + dot(dot(sig, w1aas1), e0_ref[...])

    es_b = jnp.broadcast_to(q.reshape(G, 1, F), (G, A, F)).reshape(GA, F)
    ed_b = dot(h1, ad1m)
    s = _leaky(es_b + ed_b)
    m = jnp.max(s, axis=1, keepdims=True)
    ex = jnp.exp(s - m)
    denom = dot(ex, hm) + 1e-16
    alpha_pack = ex / denom

    outs = []
    for g in range(G):
        hg = h1[g * A:(g + 1) * A, :]
        hp = jnp.concatenate([hg, hg, hg, hg], axis=0) * hm
        outs.append(dot(alpha_pack[g * A:(g + 1) * A, :], hp))
    x1 = jax.nn.relu(jnp.concatenate(outs, axis=0) + b1_ref[...])

    h2 = dot(x1, w2_ref[...])
    es2 = dot(h2, as2m)
    q2 = jnp.concatenate(_per_head_rows(es2), axis=1)
    h2a = h2.reshape(G, A, F)[:, 0:1, :].reshape(G, F)
    ed2a = dot(h2a, ad2m)
    s2 = _leaky(q2 + dot(ed2a, exp_ref[...]))
    m2 = jnp.max(s2, axis=1, keepdims=True)
    ex2 = jnp.exp(s2 - m2)
    den2 = dot(ex2, hm) + 1e-16
    a2 = ex2 / den2
    stack = jnp.concatenate(
        [a2[:, h * A:(h + 1) * A].reshape(G, 1, A) for h in range(H)],
        axis=1)
    node_w = jnp.transpose(stack, (0, 2, 1)).reshape(GA, H)
    wh2 = h2 * dot(node_w, exp_ref[...])
    agent = jax.nn.relu(jnp.sum(wh2.reshape(G, A, F), axis=1)
                        + b2_ref[...])

    mu = dot(agent, wm_ref[...]) + bm_ref[...]
    lv = dot(agent, wv_ref[...]) + bv_ref[...]
    act = dot(mu, wa_ref[...]) + ba_ref[...]
    bel = dot(agent, wb_ref[...]) + bb_ref[...]

    act_out[...] = act[None]
    mu_out[...] = mu[None]
    lv_out[...] = lv[None]
    bel_out[...] = bel[None]


@jax.jit
def kernel(signals, neighbor_actions, W1, att_src1, att_dst1, b1,
           W2, att_src2, att_dst2, b2, Wm, bm, Wv, bv, Wa, ba, Wb, bb):
    acts_r = neighbor_actions.reshape(B * A, DA)
    acts_w = neighbor_actions.reshape(B, A * DA)
    w1a, w1b = W1[:DB], W1[DB:]

    hm = jnp.asarray(np.kron(np.eye(H, dtype=np.float32),
                             np.ones((HD, HD), np.float32)))
    expd = jnp.asarray(np.kron(np.eye(H, dtype=np.float32),
                               np.ones((1, A), np.float32)))
    hsel = jnp.asarray(np.kron(np.eye(H, dtype=np.float32),
                               np.ones((HD, 1), np.float32)))
    c1 = jnp.asarray(np.kron(np.ones((A, 1), np.float32),
                             np.eye(DA, dtype=np.float32)))
    t1 = jnp.asarray(np.tile(np.kron(np.eye(A, dtype=np.float32),
                                     np.ones((DA, 1), np.float32)),
                             (1, H)))
    e0 = jnp.asarray(np.kron(np.eye(H, dtype=np.float32),
                             np.eye(1, A, dtype=np.float32)))
    eyef = jnp.asarray(np.eye(F, dtype=np.float32))

    full = lambda shape: pl.BlockSpec(shape, lambda i: (0, 0))
    tile = lambda shape: pl.BlockSpec(shape, lambda i: (i, 0))
    tile3 = lambda w: pl.BlockSpec((1, G, w), lambda i: (i, 0, 0))

    NB = B // G
    out_shapes = (
        jax.ShapeDtypeStruct((NB, G, DA * A), jnp.float32),
        jax.ShapeDtypeStruct((NB, G, LAT), jnp.float32),
        jax.ShapeDtypeStruct((NB, G, LAT), jnp.float32),
        jax.ShapeDtypeStruct((NB, G, DB), jnp.float32),
    )
    grid = (B // G,)
    res = pl.pallas_call(
        _fused_kernel,
        grid=grid,
        in_specs=[
            tile3(DB),
            tile((GA, DA)),
            tile((G, A * DA)),
            full((DB, F)),
            full((DA, F)),
            full((1, F)),
            full((1, F)),
            full((1, F)),
            full((1, F)),
            full((1, F)),
            full((F, F)),
            full((1, F)),
            full((F, LAT)),
            full((1, LAT)),
            full((F, LAT)),
            full((1, LAT)),
            full((LAT, DA * A)),
            full((1, DA * A)),
            full((F, DB)),
            full((1, DB)),
            full((F, F)),
            full((H, H * A)),
            full((F, H)),
            full((A * DA, DA)),
            full((A * DA, H * A)),
            full((H, H * A)),
            full((F, F)),
        ],
        out_specs=(
            tile3(DA * A),
            tile3(LAT),
            tile3(LAT),
            tile3(DB),
        ),
        out_shape=out_shapes,
        compiler_params=pltpu.CompilerParams(
            dimension_semantics=("parallel",),
        ),
    )(signals.reshape(NB, G, DB), acts_r, acts_w, w1a, w1b,
      att_src1.reshape(1, F), att_dst1.reshape(1, F),
      att_src2.reshape(1, F), att_dst2.reshape(1, F),
      b1.reshape(1, F), W2, b2.reshape(1, F), Wm,
      bm.reshape(1, LAT), Wv, bv.reshape(1, LAT), Wa,
      ba.reshape(1, DA * A), Wb, bb.reshape(1, DB), hm, expd, hsel,
      c1, t1, e0, eyef)
    act, mu, lv, bel = res
    return (act.reshape(B, DA * A), mu.reshape(B, LAT),
            lv.reshape(B, LAT), bel.reshape(B, DB))


# speedup vs baseline: 1.0499x; 1.0136x over previous
import functools

import numpy as np
import jax
import jax.numpy as jnp
from jax.experimental import pallas as pl
from jax.experimental.pallas import tpu as pltpu

B = 512
A = 32
DA = 8
DB = 8
H = 4
HD = 32
F = H * HD
LAT = 64

G = 512
GA = G * A


def _leaky(x):
    return jnp.where(x >= 0, x, 0.2 * x)


def _per_head_rows(es):
    es3 = es.reshape(G, A, H)
    est = jnp.transpose(es3, (0, 2, 1))
    return [est[:, h:h + 1, :].reshape(G, A) for h in range(H)]


def _fused_kernel(sig_ref, acts_ref, actsw_ref, w1a_ref, w1b_ref,
                  as1_ref, ad1_ref, as2_ref, ad2_ref,
                  b1_ref, w2_ref, b2_ref,
                  wm_ref, bm_ref, wv_ref, bv_ref, wa_ref, ba_ref,
                  wb_ref, bb_ref, hm_ref, exp_ref, hsel_ref,
                  c1_ref, t1_ref, e0_ref, eyef_ref,
                  act_out, mu_out, lv_out, bel_out):
    f32 = jnp.float32
    dot = functools.partial(jnp.dot, preferred_element_type=f32)

    sig = sig_ref[...].reshape(G, DB)
    acts = acts_ref[...]
    hm = hm_ref[...]
    hsel = hsel_ref[...]
    eyef = eyef_ref[...]

    v1 = dot(w1b_ref[...] * as1_ref[...], hsel)
    w1aas1 = dot(w1a_ref[...] * as1_ref[...], hsel)
    kq = dot(dot(c1_ref[...], v1), exp_ref[...]) * t1_ref[...]
    ad1m = dot(eyef * ad1_ref[...], hm)
    as2m = dot(eyef * as2_ref[...], hsel)
    ad2m = dot(eyef * ad2_ref[...], hsel)

    sigw = dot(sig, w1a_ref[...])
    bel_rows = jnp.concatenate(
        [sigw.reshape(G, 1, F), jnp.zeros((G, A - 1, F), f32)],
        axis=1).reshape(GA, F)
    h1 = bel_rows + dot(acts, w1b_ref[...])

    q = dot(actsw_ref[...], kq) \
        + dot(dot(sig, w1aas1), e0_ref[...])

    es_b = jnp.broadcast_to(q.reshape(G, 1, F), (G, A, F)).reshape(GA, F)
    ed_b = dot(h1, ad1m)
    s = _leaky(es_b + ed_b)
    m = jnp.max(s, axis=1, keepdims=True)
    ex = jnp.exp(s - m)
    denom = dot(ex, hm) + 1e-16
    alpha_pack = ex / denom

    outs = []
    for g in range(G):
        hg = h1[g * A:(g + 1) * A, :]
        hp = jnp.broadcast_to(hg.reshape(1, A, F),
                              (H, A, F)).reshape(F, F) * hm
        outs.append(dot(alpha_pack[g * A:(g + 1) * A, :], hp))
    x1 = jax.nn.relu(jnp.concatenate(outs, axis=0) + b1_ref[...])

    h2 = dot(x1, w2_ref[...])
    es2 = dot(h2, as2m)
    q2 = jnp.concatenate(_per_head_rows(es2), axis=1)
    ed2 = dot(h2, ad2m)
    ed2a = ed2.reshape(G, A, H)[:, 0:1, :].reshape(G, H)
    s2 = _leaky(q2 + dot(ed2a, exp_ref[...]))
    m2 = jnp.max(s2, axis=1, keepdims=True)
    ex2 = jnp.exp(s2 - m2)
    den2 = dot(ex2, hm) + 1e-16
    a2 = ex2 / den2
    stack = jnp.concatenate(
        [a2[:, h * A:(h + 1) * A].reshape(G, 1, A) for h in range(H)],
        axis=1)
    node_w = jnp.transpose(stack, (0, 2, 1)).reshape(GA, H)
    wh2 = h2 * dot(node_w, exp_ref[...])
    agent = jax.nn.relu(jnp.sum(wh2.reshape(G, A, F), axis=1)
                        + b2_ref[...])

    mu = dot(agent, wm_ref[...]) + bm_ref[...]
    lv = dot(agent, wv_ref[...]) + bv_ref[...]
    act = dot(mu, wa_ref[...]) + ba_ref[...]
    bel = dot(agent, wb_ref[...]) + bb_ref[...]

    act_out[...] = act[None]
    mu_out[...] = mu[None]
    lv_out[...] = lv[None]
    bel_out[...] = bel[None]


@jax.jit
def kernel(signals, neighbor_actions, W1, att_src1, att_dst1, b1,
           W2, att_src2, att_dst2, b2, Wm, bm, Wv, bv, Wa, ba, Wb, bb):
    acts_r = neighbor_actions.reshape(B * A, DA)
    acts_w = neighbor_actions.reshape(B, A * DA)
    w1a, w1b = W1[:DB], W1[DB:]

    hm = jnp.asarray(np.kron(np.eye(H, dtype=np.float32),
                             np.ones((HD, HD), np.float32)))
    expd = jnp.asarray(np.kron(np.eye(H, dtype=np.float32),
                               np.ones((1, A), np.float32)))
    hsel = jnp.asarray(np.kron(np.eye(H, dtype=np.float32),
                               np.ones((HD, 1), np.float32)))
    c1 = jnp.asarray(np.kron(np.ones((A, 1), np.float32),
                             np.eye(DA, dtype=np.float32)))
    t1 = jnp.asarray(np.tile(np.kron(np.eye(A, dtype=np.float32),
                                     np.ones((DA, 1), np.float32)),
                             (1, H)))
    e0 = jnp.asarray(np.kron(np.eye(H, dtype=np.float32),
                             np.eye(1, A, dtype=np.float32)))
    eyef = jnp.asarray(np.eye(F, dtype=np.float32))

    full = lambda shape: pl.BlockSpec(shape, lambda i: (0, 0))
    tile = lambda shape: pl.BlockSpec(shape, lambda i: (i, 0))
    tile3 = lambda w: pl.BlockSpec((1, G, w), lambda i: (i, 0, 0))

    NB = B // G
    out_shapes = (
        jax.ShapeDtypeStruct((NB, G, DA * A), jnp.float32),
        jax.ShapeDtypeStruct((NB, G, LAT), jnp.float32),
        jax.ShapeDtypeStruct((NB, G, LAT), jnp.float32),
        jax.ShapeDtypeStruct((NB, G, DB), jnp.float32),
    )
    grid = (B // G,)
    res = pl.pallas_call(
        _fused_kernel,
        grid=grid,
        in_specs=[
            tile3(DB),
            tile((GA, DA)),
            tile((G, A * DA)),
            full((DB, F)),
            full((DA, F)),
            full((1, F)),
            full((1, F)),
            full((1, F)),
            full((1, F)),
            full((1, F)),
            full((F, F)),
            full((1, F)),
            full((F, LAT)),
            full((1, LAT)),
            full((F, LAT)),
            full((1, LAT)),
            full((LAT, DA * A)),
            full((1, DA * A)),
            full((F, DB)),
            full((1, DB)),
            full((F, F)),
            full((H, H * A)),
            full((F, H)),
            full((A * DA, DA)),
            full((A * DA, H * A)),
            full((H, H * A)),
            full((F, F)),
        ],
        out_specs=(
            tile3(DA * A),
            tile3(LAT),
            tile3(LAT),
            tile3(DB),
        ),
        out_shape=out_shapes,
        compiler_params=pltpu.CompilerParams(
            dimension_semantics=("parallel",),
        ),
    )(signals.reshape(NB, G, DB), acts_r, acts_w, w1a, w1b,
      att_src1.reshape(1, F), att_dst1.reshape(1, F),
      att_src2.reshape(1, F), att_dst2.reshape(1, F),
      b1.reshape(1, F), W2, b2.reshape(1, F), Wm,
      bm.reshape(1, LAT), Wv, bv.reshape(1, LAT), Wa,
      ba.reshape(1, DA * A), Wb, bb.reshape(1, DB), hm, expd, hsel,
      c1, t1, e0, eyef)
    act, mu, lv, bel = res
    return (act.reshape(B, DA * A), mu.reshape(B, LAT),
            lv.reshape(B, LAT), bel.reshape(B, DB))
